# baseline (speedup 1.0000x reference)
import functools

import jax
import jax.numpy as jnp
from jax import lax
from jax.experimental import pallas as pl
from jax.experimental.pallas import tpu as pltpu
from jax.experimental.pallas import tpu_sc as plsc

NC = 2
NS = 16
NW = NC * NS
LANES = 16
BLK = 8192
GRP = 16

NB1 = 2048
NB2 = 2048
NB3 = 1024
ST1 = NB1 + 1
ST2 = NB2 + 1
ST3 = NB3 + 1


def _key16(v):
    bits = lax.bitcast_convert_type(v, jnp.int32)
    return bits ^ ((bits >> 31) & jnp.int32(0x7FFFFFFF))


def _zero(ref, nwords):
    z = jnp.zeros((LANES,), jnp.int32)

    def body(i, _):
        ref[pl.ds(i * LANES, LANES)] = z
        return 0

    lax.fori_loop(0, nwords // LANES, body, 0, unroll=8)


def _reduce_lanes(hist, red, nb, st):
    def body(j, _):
        acc = hist[pl.ds(j * LANES, LANES)]
        for l in range(1, LANES):
            acc = acc + hist[pl.ds(l * st + j * LANES, LANES)]
        red[pl.ds(j * LANES, LANES)] = acc
        return 0

    lax.fori_loop(0, nb // LANES, body, 0)


def _scan_desc(gh, coarse, nb, k_rem):
    ng = nb // LANES

    def sa(g, _):
        coarse[g] = jnp.sum(gh[pl.ds(g * LANES, LANES)])
        return 0

    lax.fori_loop(0, ng, sa, 0)

    def sb(i, c):
        s, gstar = c
        g = ng - 1 - i
        v = coarse[g]
        hit = (gstar < 0) & (s + v >= k_rem)
        s2 = jnp.where((gstar < 0) & jnp.logical_not(hit), s + v, s)
        return (s2, jnp.where(hit, g, gstar))

    s, gstar = lax.fori_loop(0, ng, sb, (jnp.int32(0), jnp.int32(-1)))

    iot = lax.iota(jnp.int32, LANES)
    vec = gh[pl.ds(gstar * LANES, LANES)]
    incl = plsc.cumsum(vec)
    total = jnp.sum(vec)
    cnt_incl = s + total - incl + vec
    mask = cnt_incl >= k_rem
    lstar = jnp.max(jnp.where(mask, iot, -1))
    bstar = gstar * LANES + lstar
    s_above = s + total - jnp.sum(jnp.where(iot == lstar, incl, 0))
    return bstar, s_above


def _slice_copy(h_hbm, hv2, semh, sid, nb):
    sl = nb // NS
    return pltpu.async_copy(h_hbm.at[pl.ds(sid * (NW * sl), NW * sl)],
                            hv2, semh)


def _hist_publish(red_v, h_hbm, wid, nb, sem):
    sl = nb // NS
    cps = [pltpu.async_copy(red_v.at[pl.ds(s * sl, sl)],
                            h_hbm.at[pl.ds(s * (NW * sl) + wid * sl, sl)],
                            sem)
           for s in range(NS)]
    for cp in cps:
        cp.wait()


def _global_hist(hv2, redsl, shared, ghv, sid, nb):
    sl = nb // NS

    def body(j, _):
        acc = hv2[pl.ds(j * LANES, LANES)]
        for r in range(1, NW):
            acc = acc + hv2[pl.ds(r * sl + j * LANES, LANES)]
        redsl[pl.ds(j * LANES, LANES)] = acc
        return 0

    lax.fori_loop(0, sl // LANES, body, 0)
    pltpu.sync_copy(redsl, shared.at[pl.ds(sid * sl, sl)])
    plsc.subcore_barrier()
    pltpu.sync_copy(shared, ghv)


def _blk(ref1d, off, lw):
    del lw
    return ref1d.at[pl.ds(off, BLK)]


def _hist_blocks(x_hbm, lw, bufs, sems, hist_v, wid, ch, binfn, st, cp0):
    lanebase = lax.iota(jnp.int32, LANES) * st
    ones = jnp.ones((LANES,), jnp.int32)
    nblk = ch // BLK
    cps = [cp0, None]
    for b in range(nblk):
        if b + 1 < nblk:
            cps[(b + 1) % 2] = pltpu.async_copy(
                _blk(x_hbm, wid * ch + (b + 1) * BLK, lw),
                bufs[(b + 1) % 2], sems[(b + 1) % 2])
        cps[b % 2].wait()
        cur = bufs[b % 2]

        def body(i, _):
            pairs = []
            for g in range(GRP):
                v = cur[pl.ds(i * (GRP * LANES) + g * LANES, LANES)]
                bin_, mask = binfn(_key16(v))
                pairs.append((lanebase + bin_, mask))
            for idx, mask in pairs:
                plsc.addupdate_scatter(hist_v, [idx], ones, mask=mask)
            return 0

        lax.fori_loop(0, BLK // (GRP * LANES), body, 0, unroll=2)


def _make_kernels(shape, k):
    H, W = shape
    N = H * W
    ch = N // NW
    lw = W.bit_length() - 1
    assert (1 << lw) == W and W % BLK == 0 and N % (NW * BLK) == 0
    i32 = jnp.int32
    f32 = jnp.float32
    mesh = plsc.VectorSubcoreMesh(
        core_axis_name="c", subcore_axis_name="s",
        num_cores=NC, num_subcores=NS)

    def ids_():
        sid = lax.axis_index("s")
        return sid * NC + lax.axis_index("c"), sid

    def first_cp(x_hbm, bufs, sems, wid):
        return pltpu.async_copy(_blk(x_hbm, wid * ch, lw), bufs[0], sems[0])

    @functools.partial(
        pl.kernel,
        out_type=jax.ShapeDtypeStruct((NW * NB1,), i32),
        mesh=mesh,
        compiler_params=pltpu.CompilerParams(needs_layout_passes=False),
        scratch_types=[
            pltpu.VMEM((BLK,), f32),
            pltpu.VMEM((BLK,), f32),
            pltpu.VMEM((LANES * ST1,), i32),
            pltpu.VMEM((NB1,), i32),
            pltpu.SemaphoreType.DMA,
            pltpu.SemaphoreType.DMA,
        ])
    def k1(x_hbm, h1_hbm, buf0, buf1, hist_v, red_v, sem0, sem1):
        wid, _ = ids_()
        bufs, sems = [buf0, buf1], [sem0, sem1]
        cp0 = first_cp(x_hbm, bufs, sems, wid)
        _zero(hist_v, LANES * ST1)
        _hist_blocks(x_hbm, lw, bufs, sems, hist_v, wid, ch,
                     lambda ks: ((ks >> 21) + 1024, None), ST1, cp0)
        _reduce_lanes(hist_v, red_v, NB1, ST1)
        _hist_publish(red_v, h1_hbm, wid, NB1, sem0)

    @functools.partial(
        pl.kernel,
        out_type=(jax.ShapeDtypeStruct((NW * NB2,), i32),
                  jax.ShapeDtypeStruct((LANES,), i32)),
        mesh=mesh,
        compiler_params=pltpu.CompilerParams(needs_layout_passes=False),
        scratch_types=[
            pltpu.VMEM((NW * (NB1 // NS),), i32),
            pltpu.VMEM((NB1 // NS,), i32),
            pltpu.VMEM_SHARED((NB1,), i32),
            pltpu.VMEM((NB1,), i32),
            pltpu.VMEM((BLK,), f32),
            pltpu.VMEM((BLK,), f32),
            pltpu.VMEM((LANES * ST2,), i32),
            pltpu.VMEM((NB2,), i32),
            pltpu.VMEM((LANES,), i32),
            pltpu.SMEM((128,), i32),
            pltpu.SemaphoreType.DMA,
            pltpu.SemaphoreType.DMA,
            pltpu.SemaphoreType.DMA,
        ])
    def k2(x_hbm, h1_hbm, h2_hbm, scal2_hbm,
           hv2, redsl, shared, ghv, buf0, buf1, hist_v, red_v, scalv,
           coarse, sem0, sem1, semh):
        wid, sid = ids_()
        bufs, sems = [buf0, buf1], [sem0, sem1]
        cp0 = first_cp(x_hbm, bufs, sems, wid)
        cph = _slice_copy(h1_hbm, hv2, semh, sid, NB1)
        _zero(hist_v, LANES * ST2)
        cph.wait()
        _global_hist(hv2, redsl, shared, ghv, sid, NB1)
        b1, s1 = _scan_desc(ghv, coarse, NB1, jnp.int32(k))
        p1 = b1 - 1024

        def binfn(ks):
            return (ks >> 10) & jnp.int32(0x7FF), (ks >> 21) == p1

        _hist_blocks(x_hbm, lw, bufs, sems, hist_v, wid, ch, binfn, ST2, cp0)
        _reduce_lanes(hist_v, red_v, NB2, ST2)
        _hist_publish(red_v, h2_hbm, wid, NB2, sem0)
        lanes = lax.iota(i32, LANES)
        scalv[...] = jnp.where(lanes == 0, p1,
                               jnp.where(lanes == 1, s1, 0)).astype(i32)

        @pl.when(wid == 0)
        def _():
            pltpu.sync_copy(scalv, scal2_hbm)

    @functools.partial(
        pl.kernel,
        out_type=(jax.ShapeDtypeStruct((NW * NB3,), i32),
                  jax.ShapeDtypeStruct((LANES,), i32)),
        mesh=mesh,
        compiler_params=pltpu.CompilerParams(needs_layout_passes=False),
        scratch_types=[
            pltpu.VMEM((NW * (NB2 // NS),), i32),
            pltpu.VMEM((NB2 // NS,), i32),
            pltpu.VMEM_SHARED((NB2,), i32),
            pltpu.VMEM((NB2,), i32),
            pltpu.VMEM((BLK,), f32),
            pltpu.VMEM((BLK,), f32),
            pltpu.VMEM((LANES * ST3,), i32),
            pltpu.VMEM((NB3,), i32),
            pltpu.VMEM((LANES,), i32),
            pltpu.SMEM((128,), i32),
            pltpu.SemaphoreType.DMA,
            pltpu.SemaphoreType.DMA,
            pltpu.SemaphoreType.DMA,
        ])
    def k3(x_hbm, h2_hbm, scal2_hbm, h3_hbm, scal3_hbm,
           hv2, redsl, shared, ghv, buf0, buf1, hist_v, red_v, scalv,
           coarse, sem0, sem1, semh):
        wid, sid = ids_()
        bufs, sems = [buf0, buf1], [sem0, sem1]
        cp0 = first_cp(x_hbm, bufs, sems, wid)
        cph = _slice_copy(h2_hbm, hv2, semh, sid, NB2)
        pltpu.sync_copy(scal2_hbm, scalv)
        sv = scalv[...]
        p1 = sv[0]
        s1 = sv[1]
        _zero(hist_v, LANES * ST3)
        cph.wait()
        _global_hist(hv2, redsl, shared, ghv, sid, NB2)
        b2, sw = _scan_desc(ghv, coarse, NB2, jnp.int32(k) - s1)
        p12 = p1 * 2048 + b2
        s2 = s1 + sw

        def binfn(ks):
            return ks & jnp.int32(0x3FF), (ks >> 10) == p12

        _hist_blocks(x_hbm, lw, bufs, sems, hist_v, wid, ch, binfn, ST3, cp0)
        _reduce_lanes(hist_v, red_v, NB3, ST3)
        _hist_publish(red_v, h3_hbm, wid, NB3, sem0)
        lanes = lax.iota(i32, LANES)
        scalv[...] = jnp.where(lanes == 0, p12,
                               jnp.where(lanes == 1, s2, 0)).astype(i32)

        @pl.when(wid == 0)
        def _():
            pltpu.sync_copy(scalv, scal3_hbm)

    @functools.partial(
        pl.kernel,
        out_type=jax.ShapeDtypeStruct((N,), f32),
        mesh=mesh,
        compiler_params=pltpu.CompilerParams(needs_layout_passes=False),
        scratch_types=[
            pltpu.VMEM((NW * (NB3 // NS),), i32),
            pltpu.VMEM((NB3 // NS,), i32),
            pltpu.VMEM_SHARED((NB3,), i32),
            pltpu.VMEM((NB3,), i32),
            pltpu.VMEM((NW * (NB3 // NS),), i32),
            pltpu.VMEM((BLK,), f32),
            pltpu.VMEM((BLK,), f32),
            pltpu.VMEM((BLK,), f32),
            pltpu.VMEM((BLK,), f32),
            pltpu.VMEM((LANES,), i32),
            pltpu.SMEM((128,), i32),
            pltpu.SemaphoreType.DMA,
            pltpu.SemaphoreType.DMA,
            pltpu.SemaphoreType.DMA,
            pltpu.SemaphoreType.DMA,
            pltpu.SemaphoreType.DMA,
        ])
    def k4(x_hbm, h3_hbm, scal3_hbm, out_hbm,
           hv2, redsl, shared, ghv, slab, buf0, buf1, ob0, ob1, scalv,
           coarse, sem0, sem1, osem0, osem1, semh):
        wid, sid = ids_()
        bufs, sems = [buf0, buf1], [sem0, sem1]
        obufs, osems = [ob0, ob1], [osem0, osem1]
        cp0 = first_cp(x_hbm, bufs, sems, wid)
        cph = _slice_copy(h3_hbm, hv2, semh, sid, NB3)
        pltpu.sync_copy(scal3_hbm, scalv)
        sv = scalv[...]
        p12 = sv[0]
        s2 = sv[1]
        cph.wait()
        _global_hist(hv2, redsl, shared, ghv, sid, NB3)
        b3, sw3 = _scan_desc(ghv, coarse, NB3, jnp.int32(k) - s2)
        tk = p12 * 1024 + b3
        need = jnp.int32(k) - (s2 + sw3)
        iot = lax.iota(jnp.int32, LANES)
        sl3 = NB3 // NS
        s3 = b3 // sl3
        j3 = b3 - s3 * sl3
        jg = (j3 // LANES) * LANES
        lane = j3 - jg
        scp = pltpu.async_copy(
            h3_hbm.at[pl.ds(s3 * (NW * sl3), NW * sl3)], slab, semh)
        scp.wait()
        e_me = jnp.int32(0)
        base = jnp.int32(0)
        for w in range(NW):
            e_w = jnp.sum(jnp.where(
                iot == lane, slab[pl.ds(w * sl3 + jg, LANES)], 0))
            e_me = jnp.where(w == wid, e_w, e_me)
            base = base + jnp.where(w < wid, e_w, 0)
        m = jnp.maximum(jnp.int32(0), jnp.minimum(need - base, e_me))
        thr = tk - jnp.where(m == e_me, 1, 0).astype(i32)

        nblk = ch // BLK
        cps = [cp0, None]
        ocps = [None, None]
        for b in range(nblk):
            if b + 1 < nblk:
                cps[(b + 1) % 2] = pltpu.async_copy(
                    _blk(x_hbm, wid * ch + (b + 1) * BLK, lw),
                    bufs[(b + 1) % 2], sems[(b + 1) % 2])
            cps[b % 2].wait()
            if b >= 2:
                ocps[b % 2].wait()
            cur = bufs[b % 2]
            ob = obufs[b % 2]

            def body(i, _):
                outs = []
                for g in range(GRP):
                    slc = pl.ds(i * (GRP * LANES) + g * LANES, LANES)
                    v = cur[slc]
                    ks = _key16(v)
                    outs.append((slc, jnp.where(ks > thr, v, 0.0)))
                for slc, o in outs:
                    ob[slc] = o
                return 0

            lax.fori_loop(0, BLK // (GRP * LANES), body, 0, unroll=2)
            ocps[b % 2] = pltpu.async_copy(
                ob, _blk(out_hbm, wid * ch + b * BLK, lw), osems[b % 2])
        ocps[(nblk - 2) % 2].wait()
        ocps[(nblk - 1) % 2].wait()

        @pl.when((m > 0) & (m < e_me))
        def _():
            running = jnp.zeros((LANES,), i32)
            for b in range(nblk):
                pltpu.sync_copy(_blk(x_hbm, wid * ch + b * BLK, lw), buf0)

                def fb(i, run):
                    v = buf0[pl.ds(i * LANES, LANES)]
                    ks = _key16(v)
                    eq = ks == tk
                    incl = plsc.cumsum(eq.astype(i32))
                    pc = plsc.all_reduce_population_count(eq)
                    sel = eq & ((incl + run) <= m)
                    ob0[pl.ds(i * LANES, LANES)] = jnp.where(
                        ks > tk, v, jnp.where(sel, v, 0.0))
                    return run + pc

                running = lax.fori_loop(0, BLK // LANES, fb, running)
                pltpu.sync_copy(ob0, _blk(out_hbm, wid * ch + b * BLK, lw))

    return k1, k2, k3, k4


def kernel(x):
    shape = x.shape
    xf = x.reshape(-1)
    N = xf.shape[0]
    k = int(0.01 * N)
    k1, k2, k3, k4 = _make_kernels(shape, k)
    h1 = k1(xf)
    h2, scal2 = k2(xf, h1)
    h3, scal3 = k3(xf, h2, scal2)
    return k4(xf, h3, scal3).reshape(shape)


# speedup vs baseline: 1.0142x; 1.0142x over previous
import functools

import jax
import jax.numpy as jnp
from jax import lax
from jax.experimental import pallas as pl
from jax.experimental.pallas import tpu as pltpu
from jax.experimental.pallas import tpu_sc as plsc

NC = 2
NS = 16
NW = NC * NS
LANES = 16
BLK = 8192
GRP = 8

NB1 = 2048
NB2 = 2048
NB3 = 1024
ST1 = NB1 + 1
ST2 = NB2 + 1
ST3 = NB3 + 1


def _key16(v):
    bits = lax.bitcast_convert_type(v, jnp.int32)
    return bits ^ ((bits >> 31) & jnp.int32(0x7FFFFFFF))


def _zero(ref, nwords):
    z = jnp.zeros((LANES,), jnp.int32)

    def body(i, _):
        ref[pl.ds(i * LANES, LANES)] = z
        return 0

    lax.fori_loop(0, nwords // LANES, body, 0, unroll=8)


def _reduce_lanes(hist, red, nb, st):
    def body(j, _):
        acc = hist[pl.ds(j * LANES, LANES)]
        for l in range(1, LANES):
            acc = acc + hist[pl.ds(l * st + j * LANES, LANES)]
        red[pl.ds(j * LANES, LANES)] = acc
        return 0

    lax.fori_loop(0, nb // LANES, body, 0)


def _scan_desc(gh, coarse, nb, k_rem):
    ng = nb // LANES

    def sa(g, _):
        coarse[g] = jnp.sum(gh[pl.ds(g * LANES, LANES)])
        return 0

    lax.fori_loop(0, ng, sa, 0)

    def sb(i, c):
        s, gstar = c
        g = ng - 1 - i
        v = coarse[g]
        hit = (gstar < 0) & (s + v >= k_rem)
        s2 = jnp.where((gstar < 0) & jnp.logical_not(hit), s + v, s)
        return (s2, jnp.where(hit, g, gstar))

    s, gstar = lax.fori_loop(0, ng, sb, (jnp.int32(0), jnp.int32(-1)))

    iot = lax.iota(jnp.int32, LANES)
    vec = gh[pl.ds(gstar * LANES, LANES)]
    incl = plsc.cumsum(vec)
    total = jnp.sum(vec)
    cnt_incl = s + total - incl + vec
    mask = cnt_incl >= k_rem
    lstar = jnp.max(jnp.where(mask, iot, -1))
    bstar = gstar * LANES + lstar
    s_above = s + total - jnp.sum(jnp.where(iot == lstar, incl, 0))
    return bstar, s_above


def _slice_copy(h_hbm, hv2, semh, sid, nb):
    sl = nb // NS
    return pltpu.async_copy(h_hbm.at[pl.ds(sid * (NW * sl), NW * sl)],
                            hv2, semh)


def _hist_publish(red_v, h_hbm, wid, nb, sem):
    sl = nb // NS
    cps = [pltpu.async_copy(red_v.at[pl.ds(s * sl, sl)],
                            h_hbm.at[pl.ds(s * (NW * sl) + wid * sl, sl)],
                            sem)
           for s in range(NS)]
    for cp in cps:
        cp.wait()


def _global_hist(hv2, redsl, shared, ghv, sid, nb):
    sl = nb // NS

    def body(j, _):
        acc = hv2[pl.ds(j * LANES, LANES)]
        for r in range(1, NW):
            acc = acc + hv2[pl.ds(r * sl + j * LANES, LANES)]
        redsl[pl.ds(j * LANES, LANES)] = acc
        return 0

    lax.fori_loop(0, sl // LANES, body, 0)
    pltpu.sync_copy(redsl, shared.at[pl.ds(sid * sl, sl)])
    plsc.subcore_barrier()
    pltpu.sync_copy(shared, ghv)


def _blk(ref1d, off, lw):
    del lw
    return ref1d.at[pl.ds(off, BLK)]


def _hist_blocks(x_hbm, lw, bufs, sems, hist_v, wid, ch, binfn, st, cp0):
    lanebase = lax.iota(jnp.int32, LANES) * st
    ones = jnp.ones((LANES,), jnp.int32)
    nblk = ch // BLK
    cps = [cp0, None]
    for b in range(nblk):
        if b + 1 < nblk:
            cps[(b + 1) % 2] = pltpu.async_copy(
                _blk(x_hbm, wid * ch + (b + 1) * BLK, lw),
                bufs[(b + 1) % 2], sems[(b + 1) % 2])
        cps[b % 2].wait()
        cur = bufs[b % 2]

        def body(i, _):
            pairs = []
            for g in range(GRP):
                v = cur[pl.ds(i * (GRP * LANES) + g * LANES, LANES)]
                bin_, mask = binfn(_key16(v))
                pairs.append((lanebase + bin_, mask))
            for idx, mask in pairs:
                plsc.addupdate_scatter(hist_v, [idx], ones, mask=mask)
            return 0

        lax.fori_loop(0, BLK // (GRP * LANES), body, 0, unroll=2)


def _make_kernels(shape, k):
    H, W = shape
    N = H * W
    ch = N // NW
    lw = W.bit_length() - 1
    assert (1 << lw) == W and W % BLK == 0 and N % (NW * BLK) == 0
    i32 = jnp.int32
    f32 = jnp.float32
    mesh = plsc.VectorSubcoreMesh(
        core_axis_name="c", subcore_axis_name="s",
        num_cores=NC, num_subcores=NS)

    def ids_():
        sid = lax.axis_index("s")
        return sid * NC + lax.axis_index("c"), sid

    def first_cp(x_hbm, bufs, sems, wid):
        return pltpu.async_copy(_blk(x_hbm, wid * ch, lw), bufs[0], sems[0])

    @functools.partial(
        pl.kernel,
        out_type=jax.ShapeDtypeStruct((NW * NB1,), i32),
        mesh=mesh,
        compiler_params=pltpu.CompilerParams(needs_layout_passes=False),
        scratch_types=[
            pltpu.VMEM((BLK,), f32),
            pltpu.VMEM((BLK,), f32),
            pltpu.VMEM((LANES * ST1,), i32),
            pltpu.VMEM((NB1,), i32),
            pltpu.SemaphoreType.DMA,
            pltpu.SemaphoreType.DMA,
        ])
    def k1(x_hbm, h1_hbm, buf0, buf1, hist_v, red_v, sem0, sem1):
        wid, _ = ids_()
        bufs, sems = [buf0, buf1], [sem0, sem1]
        cp0 = first_cp(x_hbm, bufs, sems, wid)
        _zero(hist_v, LANES * ST1)
        _hist_blocks(x_hbm, lw, bufs, sems, hist_v, wid, ch,
                     lambda ks: ((ks >> 21) + 1024, None), ST1, cp0)
        _reduce_lanes(hist_v, red_v, NB1, ST1)
        _hist_publish(red_v, h1_hbm, wid, NB1, sem0)

    @functools.partial(
        pl.kernel,
        out_type=(jax.ShapeDtypeStruct((NW * NB2,), i32),
                  jax.ShapeDtypeStruct((LANES,), i32)),
        mesh=mesh,
        compiler_params=pltpu.CompilerParams(needs_layout_passes=False),
        scratch_types=[
            pltpu.VMEM((NW * (NB1 // NS),), i32),
            pltpu.VMEM((NB1 // NS,), i32),
            pltpu.VMEM_SHARED((NB1,), i32),
            pltpu.VMEM((NB1,), i32),
            pltpu.VMEM((BLK,), f32),
            pltpu.VMEM((BLK,), f32),
            pltpu.VMEM((LANES * ST2,), i32),
            pltpu.VMEM((NB2,), i32),
            pltpu.VMEM((LANES,), i32),
            pltpu.SMEM((128,), i32),
            pltpu.SemaphoreType.DMA,
            pltpu.SemaphoreType.DMA,
            pltpu.SemaphoreType.DMA,
        ])
    def k2(x_hbm, h1_hbm, h2_hbm, scal2_hbm,
           hv2, redsl, shared, ghv, buf0, buf1, hist_v, red_v, scalv,
           coarse, sem0, sem1, semh):
        wid, sid = ids_()
        bufs, sems = [buf0, buf1], [sem0, sem1]
        cp0 = first_cp(x_hbm, bufs, sems, wid)
        cph = _slice_copy(h1_hbm, hv2, semh, sid, NB1)
        _zero(hist_v, LANES * ST2)
        cph.wait()
        _global_hist(hv2, redsl, shared, ghv, sid, NB1)
        b1, s1 = _scan_desc(ghv, coarse, NB1, jnp.int32(k))
        p1 = b1 - 1024

        def binfn(ks):
            return (ks >> 10) & jnp.int32(0x7FF), (ks >> 21) == p1

        _hist_blocks(x_hbm, lw, bufs, sems, hist_v, wid, ch, binfn, ST2, cp0)
        _reduce_lanes(hist_v, red_v, NB2, ST2)
        _hist_publish(red_v, h2_hbm, wid, NB2, sem0)
        lanes = lax.iota(i32, LANES)
        scalv[...] = jnp.where(lanes == 0, p1,
                               jnp.where(lanes == 1, s1, 0)).astype(i32)

        @pl.when(wid == 0)
        def _():
            pltpu.sync_copy(scalv, scal2_hbm)

    @functools.partial(
        pl.kernel,
        out_type=(jax.ShapeDtypeStruct((NW * NB3,), i32),
                  jax.ShapeDtypeStruct((LANES,), i32)),
        mesh=mesh,
        compiler_params=pltpu.CompilerParams(needs_layout_passes=False),
        scratch_types=[
            pltpu.VMEM((NW * (NB2 // NS),), i32),
            pltpu.VMEM((NB2 // NS,), i32),
            pltpu.VMEM_SHARED((NB2,), i32),
            pltpu.VMEM((NB2,), i32),
            pltpu.VMEM((BLK,), f32),
            pltpu.VMEM((BLK,), f32),
            pltpu.VMEM((LANES * ST3,), i32),
            pltpu.VMEM((NB3,), i32),
            pltpu.VMEM((LANES,), i32),
            pltpu.SMEM((128,), i32),
            pltpu.SemaphoreType.DMA,
            pltpu.SemaphoreType.DMA,
            pltpu.SemaphoreType.DMA,
        ])
    def k3(x_hbm, h2_hbm, scal2_hbm, h3_hbm, scal3_hbm,
           hv2, redsl, shared, ghv, buf0, buf1, hist_v, red_v, scalv,
           coarse, sem0, sem1, semh):
        wid, sid = ids_()
        bufs, sems = [buf0, buf1], [sem0, sem1]
        cp0 = first_cp(x_hbm, bufs, sems, wid)
        cph = _slice_copy(h2_hbm, hv2, semh, sid, NB2)
        pltpu.sync_copy(scal2_hbm, scalv)
        sv = scalv[...]
        p1 = sv[0]
        s1 = sv[1]
        _zero(hist_v, LANES * ST3)
        cph.wait()
        _global_hist(hv2, redsl, shared, ghv, sid, NB2)
        b2, sw = _scan_desc(ghv, coarse, NB2, jnp.int32(k) - s1)
        p12 = p1 * 2048 + b2
        s2 = s1 + sw

        def binfn(ks):
            return ks & jnp.int32(0x3FF), (ks >> 10) == p12

        _hist_blocks(x_hbm, lw, bufs, sems, hist_v, wid, ch, binfn, ST3, cp0)
        _reduce_lanes(hist_v, red_v, NB3, ST3)
        _hist_publish(red_v, h3_hbm, wid, NB3, sem0)
        lanes = lax.iota(i32, LANES)
        scalv[...] = jnp.where(lanes == 0, p12,
                               jnp.where(lanes == 1, s2, 0)).astype(i32)

        @pl.when(wid == 0)
        def _():
            pltpu.sync_copy(scalv, scal3_hbm)

    @functools.partial(
        pl.kernel,
        out_type=jax.ShapeDtypeStruct((N,), f32),
        mesh=mesh,
        compiler_params=pltpu.CompilerParams(needs_layout_passes=False),
        scratch_types=[
            pltpu.VMEM((NW * (NB3 // NS),), i32),
            pltpu.VMEM((NB3 // NS,), i32),
            pltpu.VMEM_SHARED((NB3,), i32),
            pltpu.VMEM((NB3,), i32),
            pltpu.VMEM((NW * (NB3 // NS),), i32),
            pltpu.VMEM((BLK,), f32),
            pltpu.VMEM((BLK,), f32),
            pltpu.VMEM((BLK,), f32),
            pltpu.VMEM((BLK,), f32),
            pltpu.VMEM((LANES,), i32),
            pltpu.SMEM((128,), i32),
            pltpu.SemaphoreType.DMA,
            pltpu.SemaphoreType.DMA,
            pltpu.SemaphoreType.DMA,
            pltpu.SemaphoreType.DMA,
            pltpu.SemaphoreType.DMA,
        ])
    def k4(x_hbm, h3_hbm, scal3_hbm, out_hbm,
           hv2, redsl, shared, ghv, slab, buf0, buf1, ob0, ob1, scalv,
           coarse, sem0, sem1, osem0, osem1, semh):
        wid, sid = ids_()
        bufs, sems = [buf0, buf1], [sem0, sem1]
        obufs, osems = [ob0, ob1], [osem0, osem1]
        cp0 = first_cp(x_hbm, bufs, sems, wid)
        cph = _slice_copy(h3_hbm, hv2, semh, sid, NB3)
        pltpu.sync_copy(scal3_hbm, scalv)
        sv = scalv[...]
        p12 = sv[0]
        s2 = sv[1]
        cph.wait()
        _global_hist(hv2, redsl, shared, ghv, sid, NB3)
        b3, sw3 = _scan_desc(ghv, coarse, NB3, jnp.int32(k) - s2)
        tk = p12 * 1024 + b3
        need = jnp.int32(k) - (s2 + sw3)
        iot = lax.iota(jnp.int32, LANES)
        sl3 = NB3 // NS
        s3 = b3 // sl3
        j3 = b3 - s3 * sl3
        jg = (j3 // LANES) * LANES
        lane = j3 - jg
        scp = pltpu.async_copy(
            h3_hbm.at[pl.ds(s3 * (NW * sl3), NW * sl3)], slab, semh)
        scp.wait()
        e_me = jnp.int32(0)
        base = jnp.int32(0)
        for w in range(NW):
            e_w = jnp.sum(jnp.where(
                iot == lane, slab[pl.ds(w * sl3 + jg, LANES)], 0))
            e_me = jnp.where(w == wid, e_w, e_me)
            base = base + jnp.where(w < wid, e_w, 0)
        m = jnp.maximum(jnp.int32(0), jnp.minimum(need - base, e_me))
        thr = tk - jnp.where(m == e_me, 1, 0).astype(i32)
        thr_bits = jnp.where(thr >= 0, thr, thr ^ jnp.int32(0x7FFFFFFF))
        f_thr = lax.bitcast_convert_type(thr_bits, f32)

        nblk = ch // BLK
        cps = [cp0, None]
        ocps = [None, None]
        for b in range(nblk):
            if b + 1 < nblk:
                cps[(b + 1) % 2] = pltpu.async_copy(
                    _blk(x_hbm, wid * ch + (b + 1) * BLK, lw),
                    bufs[(b + 1) % 2], sems[(b + 1) % 2])
            cps[b % 2].wait()
            if b >= 2:
                ocps[b % 2].wait()
            cur = bufs[b % 2]
            ob = obufs[b % 2]

            def body(i, _):
                outs = []
                for g in range(GRP):
                    slc = pl.ds(i * (GRP * LANES) + g * LANES, LANES)
                    v = cur[slc]
                    outs.append((slc, jnp.where(v > f_thr, v, 0.0)))
                for slc, o in outs:
                    ob[slc] = o
                return 0

            lax.fori_loop(0, BLK // (GRP * LANES), body, 0, unroll=2)
            ocps[b % 2] = pltpu.async_copy(
                ob, _blk(out_hbm, wid * ch + b * BLK, lw), osems[b % 2])
        ocps[(nblk - 2) % 2].wait()
        ocps[(nblk - 1) % 2].wait()

        @pl.when((m > 0) & (m < e_me))
        def _():
            running = jnp.zeros((LANES,), i32)
            for b in range(nblk):
                pltpu.sync_copy(_blk(x_hbm, wid * ch + b * BLK, lw), buf0)

                def fb(i, run):
                    v = buf0[pl.ds(i * LANES, LANES)]
                    ks = _key16(v)
                    eq = ks == tk
                    incl = plsc.cumsum(eq.astype(i32))
                    pc = plsc.all_reduce_population_count(eq)
                    sel = eq & ((incl + run) <= m)
                    ob0[pl.ds(i * LANES, LANES)] = jnp.where(
                        ks > tk, v, jnp.where(sel, v, 0.0))
                    return run + pc

                running = lax.fori_loop(0, BLK // LANES, fb, running)
                pltpu.sync_copy(ob0, _blk(out_hbm, wid * ch + b * BLK, lw))

    return k1, k2, k3, k4


def kernel(x):
    shape = x.shape
    xf = x.reshape(-1)
    N = xf.shape[0]
    k = int(0.01 * N)
    k1, k2, k3, k4 = _make_kernels(shape, k)
    h1 = k1(xf)
    h2, scal2 = k2(xf, h1)
    h3, scal3 = k3(xf, h2, scal2)
    return k4(xf, h3, scal3).reshape(shape)


# speedup vs baseline: 1.0221x; 1.0078x over previous
import functools

import jax
import jax.numpy as jnp
from jax import lax
from jax.experimental import pallas as pl
from jax.experimental.pallas import tpu as pltpu
from jax.experimental.pallas import tpu_sc as plsc

NC = 2
NS = 16
NW = NC * NS
LANES = 16
BLK = 8192
GRP = 8

NB1 = 2048
NB2 = 2048
NB3 = 1024
ST1 = NB1 + 1
ST2 = NB2 + 1
ST3 = NB3 + 1


def _key16(v):
    bits = lax.bitcast_convert_type(v, jnp.int32)
    return bits ^ ((bits >> 31) & jnp.int32(0x7FFFFFFF))


def _zero(ref, nwords):
    z = jnp.zeros((LANES,), jnp.int32)

    def body(i, _):
        ref[pl.ds(i * LANES, LANES)] = z
        return 0

    lax.fori_loop(0, nwords // LANES, body, 0, unroll=8)


def _reduce_lanes(hist, red, nb, st):
    def body(j, _):
        acc = hist[pl.ds(j * LANES, LANES)]
        for l in range(1, LANES):
            acc = acc + hist[pl.ds(l * st + j * LANES, LANES)]
        red[pl.ds(j * LANES, LANES)] = acc
        return 0

    lax.fori_loop(0, nb // LANES, body, 0)


def _scan_desc(gh, coarse, nb, k_rem):
    ng = nb // LANES

    def sa(g, _):
        coarse[g] = jnp.sum(gh[pl.ds(g * LANES, LANES)])
        return 0

    lax.fori_loop(0, ng, sa, 0)

    def sb(i, c):
        s, gstar = c
        g = ng - 1 - i
        v = coarse[g]
        hit = (gstar < 0) & (s + v >= k_rem)
        s2 = jnp.where((gstar < 0) & jnp.logical_not(hit), s + v, s)
        return (s2, jnp.where(hit, g, gstar))

    s, gstar = lax.fori_loop(0, ng, sb, (jnp.int32(0), jnp.int32(-1)))

    iot = lax.iota(jnp.int32, LANES)
    vec = gh[pl.ds(gstar * LANES, LANES)]
    incl = plsc.cumsum(vec)
    total = jnp.sum(vec)
    cnt_incl = s + total - incl + vec
    mask = cnt_incl >= k_rem
    lstar = jnp.max(jnp.where(mask, iot, -1))
    bstar = gstar * LANES + lstar
    s_above = s + total - jnp.sum(jnp.where(iot == lstar, incl, 0))
    return bstar, s_above


def _slice_copy(h_hbm, hv2, semh, sid, nb):
    sl = nb // NS
    return pltpu.async_copy(h_hbm.at[pl.ds(sid * (NW * sl), NW * sl)],
                            hv2, semh)


def _hist_publish(red_v, h_hbm, wid, nb, sem):
    sl = nb // NS
    cps = [pltpu.async_copy(red_v.at[pl.ds(s * sl, sl)],
                            h_hbm.at[pl.ds(s * (NW * sl) + wid * sl, sl)],
                            sem)
           for s in range(NS)]
    for cp in cps:
        cp.wait()


def _global_hist(hv2, redsl, shared, ghv, sid, nb):
    sl = nb // NS

    def body(j, _):
        acc = hv2[pl.ds(j * LANES, LANES)]
        for r in range(1, NW):
            acc = acc + hv2[pl.ds(r * sl + j * LANES, LANES)]
        redsl[pl.ds(j * LANES, LANES)] = acc
        return 0

    lax.fori_loop(0, sl // LANES, body, 0)
    pltpu.sync_copy(redsl, shared.at[pl.ds(sid * sl, sl)])
    plsc.subcore_barrier()
    pltpu.sync_copy(shared, ghv)


def _blk(ref1d, off, lw):
    del lw
    return ref1d.at[pl.ds(off, BLK)]


def _hist_blocks(x_hbm, lw, bufs, sems, hist_v, wid, ch, binfn, st, cp0,
                 bias=0):
    lanebase = lax.iota(jnp.int32, LANES) * st + bias
    ones = jnp.ones((LANES,), jnp.int32)
    nblk = ch // BLK
    cps = [cp0, None]
    for b in range(nblk):
        if b + 1 < nblk:
            cps[(b + 1) % 2] = pltpu.async_copy(
                _blk(x_hbm, wid * ch + (b + 1) * BLK, lw),
                bufs[(b + 1) % 2], sems[(b + 1) % 2])
        cps[b % 2].wait()
        cur = bufs[b % 2]

        def body(i, _):
            pairs = []
            for g in range(GRP):
                v = cur[pl.ds(i * (GRP * LANES) + g * LANES, LANES)]
                bin_, mask = binfn(_key16(v))
                pairs.append((lanebase + bin_, mask))
            for idx, mask in pairs:
                plsc.addupdate_scatter(hist_v, [idx], ones, mask=mask)
            return 0

        lax.fori_loop(0, BLK // (GRP * LANES), body, 0, unroll=2)


def _make_kernels(shape, k):
    H, W = shape
    N = H * W
    ch = N // NW
    lw = W.bit_length() - 1
    assert (1 << lw) == W and W % BLK == 0 and N % (NW * BLK) == 0
    i32 = jnp.int32
    f32 = jnp.float32
    mesh = plsc.VectorSubcoreMesh(
        core_axis_name="c", subcore_axis_name="s",
        num_cores=NC, num_subcores=NS)

    def ids_():
        sid = lax.axis_index("s")
        return sid * NC + lax.axis_index("c"), sid

    def first_cp(x_hbm, bufs, sems, wid):
        return pltpu.async_copy(_blk(x_hbm, wid * ch, lw), bufs[0], sems[0])

    @functools.partial(
        pl.kernel,
        out_type=jax.ShapeDtypeStruct((NW * NB1,), i32),
        mesh=mesh,
        compiler_params=pltpu.CompilerParams(needs_layout_passes=False),
        scratch_types=[
            pltpu.VMEM((BLK,), f32),
            pltpu.VMEM((BLK,), f32),
            pltpu.VMEM((LANES * ST1,), i32),
            pltpu.VMEM((NB1,), i32),
            pltpu.SemaphoreType.DMA,
            pltpu.SemaphoreType.DMA,
        ])
    def k1(x_hbm, h1_hbm, buf0, buf1, hist_v, red_v, sem0, sem1):
        wid, _ = ids_()
        bufs, sems = [buf0, buf1], [sem0, sem1]
        cp0 = first_cp(x_hbm, bufs, sems, wid)
        _zero(hist_v, LANES * ST1)
        _hist_blocks(x_hbm, lw, bufs, sems, hist_v, wid, ch,
                     lambda ks: (ks >> 21, None), ST1, cp0, bias=1024)
        _reduce_lanes(hist_v, red_v, NB1, ST1)
        _hist_publish(red_v, h1_hbm, wid, NB1, sem0)

    @functools.partial(
        pl.kernel,
        out_type=(jax.ShapeDtypeStruct((NW * NB2,), i32),
                  jax.ShapeDtypeStruct((LANES,), i32)),
        mesh=mesh,
        compiler_params=pltpu.CompilerParams(needs_layout_passes=False),
        scratch_types=[
            pltpu.VMEM((NW * (NB1 // NS),), i32),
            pltpu.VMEM((NB1 // NS,), i32),
            pltpu.VMEM_SHARED((NB1,), i32),
            pltpu.VMEM((NB1,), i32),
            pltpu.VMEM((BLK,), f32),
            pltpu.VMEM((BLK,), f32),
            pltpu.VMEM((LANES * ST2,), i32),
            pltpu.VMEM((NB2,), i32),
            pltpu.VMEM((LANES,), i32),
            pltpu.SMEM((128,), i32),
            pltpu.SemaphoreType.DMA,
            pltpu.SemaphoreType.DMA,
            pltpu.SemaphoreType.DMA,
        ])
    def k2(x_hbm, h1_hbm, h2_hbm, scal2_hbm,
           hv2, redsl, shared, ghv, buf0, buf1, hist_v, red_v, scalv,
           coarse, sem0, sem1, semh):
        wid, sid = ids_()
        bufs, sems = [buf0, buf1], [sem0, sem1]
        cp0 = first_cp(x_hbm, bufs, sems, wid)
        cph = _slice_copy(h1_hbm, hv2, semh, sid, NB1)
        _zero(hist_v, LANES * ST2)
        cph.wait()
        _global_hist(hv2, redsl, shared, ghv, sid, NB1)
        b1, s1 = _scan_desc(ghv, coarse, NB1, jnp.int32(k))
        p1 = b1 - 1024

        def binfn(ks):
            return ks >> 10, (ks >> 21) == p1

        _hist_blocks(x_hbm, lw, bufs, sems, hist_v, wid, ch, binfn, ST2,
                     cp0, bias=-p1 * 2048)
        _reduce_lanes(hist_v, red_v, NB2, ST2)
        _hist_publish(red_v, h2_hbm, wid, NB2, sem0)
        lanes = lax.iota(i32, LANES)
        scalv[...] = jnp.where(lanes == 0, p1,
                               jnp.where(lanes == 1, s1, 0)).astype(i32)

        @pl.when(wid == 0)
        def _():
            pltpu.sync_copy(scalv, scal2_hbm)

    @functools.partial(
        pl.kernel,
        out_type=(jax.ShapeDtypeStruct((NW * NB3,), i32),
                  jax.ShapeDtypeStruct((LANES,), i32)),
        mesh=mesh,
        compiler_params=pltpu.CompilerParams(needs_layout_passes=False),
        scratch_types=[
            pltpu.VMEM((NW * (NB2 // NS),), i32),
            pltpu.VMEM((NB2 // NS,), i32),
            pltpu.VMEM_SHARED((NB2,), i32),
            pltpu.VMEM((NB2,), i32),
            pltpu.VMEM((BLK,), f32),
            pltpu.VMEM((BLK,), f32),
            pltpu.VMEM((LANES * ST3,), i32),
            pltpu.VMEM((NB3,), i32),
            pltpu.VMEM((LANES,), i32),
            pltpu.SMEM((128,), i32),
            pltpu.SemaphoreType.DMA,
            pltpu.SemaphoreType.DMA,
            pltpu.SemaphoreType.DMA,
        ])
    def k3(x_hbm, h2_hbm, scal2_hbm, h3_hbm, scal3_hbm,
           hv2, redsl, shared, ghv, buf0, buf1, hist_v, red_v, scalv,
           coarse, sem0, sem1, semh):
        wid, sid = ids_()
        bufs, sems = [buf0, buf1], [sem0, sem1]
        cp0 = first_cp(x_hbm, bufs, sems, wid)
        cph = _slice_copy(h2_hbm, hv2, semh, sid, NB2)
        pltpu.sync_copy(scal2_hbm, scalv)
        sv = scalv[...]
        p1 = sv[0]
        s1 = sv[1]
        _zero(hist_v, LANES * ST3)
        cph.wait()
        _global_hist(hv2, redsl, shared, ghv, sid, NB2)
        b2, sw = _scan_desc(ghv, coarse, NB2, jnp.int32(k) - s1)
        p12 = p1 * 2048 + b2
        s2 = s1 + sw

        def binfn(ks):
            return ks, (ks >> 10) == p12

        _hist_blocks(x_hbm, lw, bufs, sems, hist_v, wid, ch, binfn, ST3,
                     cp0, bias=-p12 * 1024)
        _reduce_lanes(hist_v, red_v, NB3, ST3)
        _hist_publish(red_v, h3_hbm, wid, NB3, sem0)
        lanes = lax.iota(i32, LANES)
        scalv[...] = jnp.where(lanes == 0, p12,
                               jnp.where(lanes == 1, s2, 0)).astype(i32)

        @pl.when(wid == 0)
        def _():
            pltpu.sync_copy(scalv, scal3_hbm)

    @functools.partial(
        pl.kernel,
        out_type=jax.ShapeDtypeStruct((N,), f32),
        mesh=mesh,
        compiler_params=pltpu.CompilerParams(needs_layout_passes=False),
        scratch_types=[
            pltpu.VMEM((NW * (NB3 // NS),), i32),
            pltpu.VMEM((NB3 // NS,), i32),
            pltpu.VMEM_SHARED((NB3,), i32),
            pltpu.VMEM((NB3,), i32),
            pltpu.VMEM((NW * (NB3 // NS),), i32),
            pltpu.VMEM((BLK,), f32),
            pltpu.VMEM((BLK,), f32),
            pltpu.VMEM((BLK,), f32),
            pltpu.VMEM((BLK,), f32),
            pltpu.VMEM((LANES,), i32),
            pltpu.SMEM((128,), i32),
            pltpu.SemaphoreType.DMA,
            pltpu.SemaphoreType.DMA,
            pltpu.SemaphoreType.DMA,
            pltpu.SemaphoreType.DMA,
            pltpu.SemaphoreType.DMA,
        ])
    def k4(x_hbm, h3_hbm, scal3_hbm, out_hbm,
           hv2, redsl, shared, ghv, slab, buf0, buf1, ob0, ob1, scalv,
           coarse, sem0, sem1, osem0, osem1, semh):
        wid, sid = ids_()
        bufs, sems = [buf0, buf1], [sem0, sem1]
        obufs, osems = [ob0, ob1], [osem0, osem1]
        cp0 = first_cp(x_hbm, bufs, sems, wid)
        cph = _slice_copy(h3_hbm, hv2, semh, sid, NB3)
        pltpu.sync_copy(scal3_hbm, scalv)
        sv = scalv[...]
        p12 = sv[0]
        s2 = sv[1]
        cph.wait()
        _global_hist(hv2, redsl, shared, ghv, sid, NB3)
        b3, sw3 = _scan_desc(ghv, coarse, NB3, jnp.int32(k) - s2)
        tk = p12 * 1024 + b3
        need = jnp.int32(k) - (s2 + sw3)
        iot = lax.iota(jnp.int32, LANES)
        sl3 = NB3 // NS
        s3 = b3 // sl3
        j3 = b3 - s3 * sl3
        jg = (j3 // LANES) * LANES
        lane = j3 - jg
        scp = pltpu.async_copy(
            h3_hbm.at[pl.ds(s3 * (NW * sl3), NW * sl3)], slab, semh)
        scp.wait()
        e_me = jnp.int32(0)
        base = jnp.int32(0)
        for w in range(NW):
            e_w = jnp.sum(jnp.where(
                iot == lane, slab[pl.ds(w * sl3 + jg, LANES)], 0))
            e_me = jnp.where(w == wid, e_w, e_me)
            base = base + jnp.where(w < wid, e_w, 0)
        m = jnp.maximum(jnp.int32(0), jnp.minimum(need - base, e_me))
        thr = tk - jnp.where(m == e_me, 1, 0).astype(i32)
        thr_bits = jnp.where(thr >= 0, thr, thr ^ jnp.int32(0x7FFFFFFF))
        f_thr = lax.bitcast_convert_type(thr_bits, f32)

        nblk = ch // BLK
        cps = [cp0, None]
        ocps = [None, None]
        for b in range(nblk):
            if b + 1 < nblk:
                cps[(b + 1) % 2] = pltpu.async_copy(
                    _blk(x_hbm, wid * ch + (b + 1) * BLK, lw),
                    bufs[(b + 1) % 2], sems[(b + 1) % 2])
            cps[b % 2].wait()
            if b >= 2:
                ocps[b % 2].wait()
            cur = bufs[b % 2]
            ob = obufs[b % 2]

            def body(i, _):
                outs = []
                for g in range(GRP):
                    slc = pl.ds(i * (GRP * LANES) + g * LANES, LANES)
                    v = cur[slc]
                    outs.append((slc, jnp.where(v > f_thr, v, 0.0)))
                for slc, o in outs:
                    ob[slc] = o
                return 0

            lax.fori_loop(0, BLK // (GRP * LANES), body, 0, unroll=2)
            ocps[b % 2] = pltpu.async_copy(
                ob, _blk(out_hbm, wid * ch + b * BLK, lw), osems[b % 2])
        ocps[(nblk - 2) % 2].wait()
        ocps[(nblk - 1) % 2].wait()

        @pl.when((m > 0) & (m < e_me))
        def _():
            running = jnp.zeros((LANES,), i32)
            for b in range(nblk):
                pltpu.sync_copy(_blk(x_hbm, wid * ch + b * BLK, lw), buf0)

                def fb(i, run):
                    v = buf0[pl.ds(i * LANES, LANES)]
                    ks = _key16(v)
                    eq = ks == tk
                    incl = plsc.cumsum(eq.astype(i32))
                    pc = plsc.all_reduce_population_count(eq)
                    sel = eq & ((incl + run) <= m)
                    ob0[pl.ds(i * LANES, LANES)] = jnp.where(
                        ks > tk, v, jnp.where(sel, v, 0.0))
                    return run + pc

                running = lax.fori_loop(0, BLK // LANES, fb, running)
                pltpu.sync_copy(ob0, _blk(out_hbm, wid * ch + b * BLK, lw))

    return k1, k2, k3, k4


def kernel(x):
    shape = x.shape
    xf = x.reshape(-1)
    N = xf.shape[0]
    k = int(0.01 * N)
    k1, k2, k3, k4 = _make_kernels(shape, k)
    h1 = k1(xf)
    h2, scal2 = k2(xf, h1)
    h3, scal3 = k3(xf, h2, scal2)
    return k4(xf, h3, scal3).reshape(shape)


# speedup vs baseline: 1.0843x; 1.0609x over previous
import functools

import jax
import jax.numpy as jnp
from jax import lax
from jax.experimental import pallas as pl
from jax.experimental.pallas import tpu as pltpu
from jax.experimental.pallas import tpu_sc as plsc

NC = 2
NS = 16
NW = NC * NS
LANES = 16
BLK = 16384
GRP = 8

NB1 = 2048
NB2 = 2048
NB3 = 1024
ST1 = NB1 + 1
ST2 = NB2 + 1
ST3 = NB3 + 1


def _key16(v):
    bits = lax.bitcast_convert_type(v, jnp.int32)
    return bits ^ ((bits >> 31) & jnp.int32(0x7FFFFFFF))


def _zero(ref, nwords):
    z = jnp.zeros((LANES,), jnp.int32)

    def body(i, _):
        ref[pl.ds(i * LANES, LANES)] = z
        return 0

    lax.fori_loop(0, nwords // LANES, body, 0, unroll=8)


def _reduce_lanes(hist, red, nb, st):
    def body(j, _):
        acc = hist[pl.ds(j * LANES, LANES)]
        for l in range(1, LANES):
            acc = acc + hist[pl.ds(l * st + j * LANES, LANES)]
        red[pl.ds(j * LANES, LANES)] = acc
        return 0

    lax.fori_loop(0, nb // LANES, body, 0)


def _scan_desc(gh, coarse, nb, k_rem):
    ng = nb // LANES

    def sa(g, _):
        coarse[g] = jnp.sum(gh[pl.ds(g * LANES, LANES)])
        return 0

    lax.fori_loop(0, ng, sa, 0)

    def sb(i, c):
        s, gstar = c
        g = ng - 1 - i
        v = coarse[g]
        hit = (gstar < 0) & (s + v >= k_rem)
        s2 = jnp.where((gstar < 0) & jnp.logical_not(hit), s + v, s)
        return (s2, jnp.where(hit, g, gstar))

    s, gstar = lax.fori_loop(0, ng, sb, (jnp.int32(0), jnp.int32(-1)))

    iot = lax.iota(jnp.int32, LANES)
    vec = gh[pl.ds(gstar * LANES, LANES)]
    incl = plsc.cumsum(vec)
    total = jnp.sum(vec)
    cnt_incl = s + total - incl + vec
    mask = cnt_incl >= k_rem
    lstar = jnp.max(jnp.where(mask, iot, -1))
    bstar = gstar * LANES + lstar
    s_above = s + total - jnp.sum(jnp.where(iot == lstar, incl, 0))
    return bstar, s_above


def _slice_copy(h_hbm, hv2, semh, sid, nb):
    sl = nb // NS
    return pltpu.async_copy(h_hbm.at[pl.ds(sid * (NW * sl), NW * sl)],
                            hv2, semh)


def _hist_publish(red_v, h_hbm, wid, nb, sem):
    sl = nb // NS
    cps = [pltpu.async_copy(red_v.at[pl.ds(s * sl, sl)],
                            h_hbm.at[pl.ds(s * (NW * sl) + wid * sl, sl)],
                            sem)
           for s in range(NS)]
    for cp in cps:
        cp.wait()


def _global_hist(hv2, redsl, shared, ghv, sid, nb):
    sl = nb // NS

    def body(j, _):
        acc = hv2[pl.ds(j * LANES, LANES)]
        for r in range(1, NW):
            acc = acc + hv2[pl.ds(r * sl + j * LANES, LANES)]
        redsl[pl.ds(j * LANES, LANES)] = acc
        return 0

    lax.fori_loop(0, sl // LANES, body, 0)
    pltpu.sync_copy(redsl, shared.at[pl.ds(sid * sl, sl)])
    plsc.subcore_barrier()
    pltpu.sync_copy(shared, ghv)


def _blk(ref1d, off, lw):
    del lw
    return ref1d.at[pl.ds(off, BLK)]


def _hist_blocks(x_hbm, lw, bufs, sems, hist_v, wid, ch, binfn, st, cp0,
                 bias=0):
    lanebase = lax.iota(jnp.int32, LANES) * st + bias
    ones = jnp.ones((LANES,), jnp.int32)
    nblk = ch // BLK
    cps = [cp0, None]
    for b in range(nblk):
        if b + 1 < nblk:
            cps[(b + 1) % 2] = pltpu.async_copy(
                _blk(x_hbm, wid * ch + (b + 1) * BLK, lw),
                bufs[(b + 1) % 2], sems[(b + 1) % 2])
        cps[b % 2].wait()
        cur = bufs[b % 2]

        def body(i, _):
            pairs = []
            for g in range(GRP):
                v = cur[pl.ds(i * (GRP * LANES) + g * LANES, LANES)]
                bin_, mask = binfn(_key16(v))
                pairs.append((lanebase + bin_, mask))
            for idx, mask in pairs:
                plsc.addupdate_scatter(hist_v, [idx], ones, mask=mask)
            return 0

        lax.fori_loop(0, BLK // (GRP * LANES), body, 0, unroll=2)


def _make_kernels(shape, k):
    H, W = shape
    N = H * W
    ch = N // NW
    lw = W.bit_length() - 1
    assert (1 << lw) == W and W % BLK == 0 and N % (NW * BLK) == 0
    i32 = jnp.int32
    f32 = jnp.float32
    mesh = plsc.VectorSubcoreMesh(
        core_axis_name="c", subcore_axis_name="s",
        num_cores=NC, num_subcores=NS)

    def ids_():
        sid = lax.axis_index("s")
        return sid * NC + lax.axis_index("c"), sid

    def first_cp(x_hbm, bufs, sems, wid):
        return pltpu.async_copy(_blk(x_hbm, wid * ch, lw), bufs[0], sems[0])

    @functools.partial(
        pl.kernel,
        out_type=jax.ShapeDtypeStruct((NW * NB1,), i32),
        mesh=mesh,
        compiler_params=pltpu.CompilerParams(needs_layout_passes=False),
        scratch_types=[
            pltpu.VMEM((BLK,), f32),
            pltpu.VMEM((BLK,), f32),
            pltpu.VMEM((LANES * ST1,), i32),
            pltpu.VMEM((NB1,), i32),
            pltpu.SemaphoreType.DMA,
            pltpu.SemaphoreType.DMA,
        ])
    def k1(x_hbm, h1_hbm, buf0, buf1, hist_v, red_v, sem0, sem1):
        wid, _ = ids_()
        bufs, sems = [buf0, buf1], [sem0, sem1]
        cp0 = first_cp(x_hbm, bufs, sems, wid)
        _zero(hist_v, LANES * ST1)
        _hist_blocks(x_hbm, lw, bufs, sems, hist_v, wid, ch,
                     lambda ks: (ks >> 21, None), ST1, cp0, bias=1024)
        _reduce_lanes(hist_v, red_v, NB1, ST1)
        _hist_publish(red_v, h1_hbm, wid, NB1, sem0)

    @functools.partial(
        pl.kernel,
        out_type=(jax.ShapeDtypeStruct((NW * NB2,), i32),
                  jax.ShapeDtypeStruct((LANES,), i32)),
        mesh=mesh,
        compiler_params=pltpu.CompilerParams(needs_layout_passes=False),
        scratch_types=[
            pltpu.VMEM((NW * (NB1 // NS),), i32),
            pltpu.VMEM((NB1 // NS,), i32),
            pltpu.VMEM_SHARED((NB1,), i32),
            pltpu.VMEM((NB1,), i32),
            pltpu.VMEM((BLK,), f32),
            pltpu.VMEM((BLK,), f32),
            pltpu.VMEM((LANES * ST2,), i32),
            pltpu.VMEM((NB2,), i32),
            pltpu.VMEM((LANES,), i32),
            pltpu.SMEM((128,), i32),
            pltpu.SemaphoreType.DMA,
            pltpu.SemaphoreType.DMA,
            pltpu.SemaphoreType.DMA,
        ])
    def k2(x_hbm, h1_hbm, h2_hbm, scal2_hbm,
           hv2, redsl, shared, ghv, buf0, buf1, hist_v, red_v, scalv,
           coarse, sem0, sem1, semh):
        wid, sid = ids_()
        bufs, sems = [buf0, buf1], [sem0, sem1]
        cp0 = first_cp(x_hbm, bufs, sems, wid)
        cph = _slice_copy(h1_hbm, hv2, semh, sid, NB1)
        _zero(hist_v, LANES * ST2)
        cph.wait()
        _global_hist(hv2, redsl, shared, ghv, sid, NB1)
        b1, s1 = _scan_desc(ghv, coarse, NB1, jnp.int32(k))
        p1 = b1 - 1024

        def binfn(ks):
            return ks >> 10, (ks >> 21) == p1

        _hist_blocks(x_hbm, lw, bufs, sems, hist_v, wid, ch, binfn, ST2,
                     cp0, bias=-p1 * 2048)
        _reduce_lanes(hist_v, red_v, NB2, ST2)
        _hist_publish(red_v, h2_hbm, wid, NB2, sem0)
        lanes = lax.iota(i32, LANES)
        scalv[...] = jnp.where(lanes == 0, p1,
                               jnp.where(lanes == 1, s1, 0)).astype(i32)

        @pl.when(wid == 0)
        def _():
            pltpu.sync_copy(scalv, scal2_hbm)

    @functools.partial(
        pl.kernel,
        out_type=(jax.ShapeDtypeStruct((NW * NB3,), i32),
                  jax.ShapeDtypeStruct((LANES,), i32)),
        mesh=mesh,
        compiler_params=pltpu.CompilerParams(needs_layout_passes=False),
        scratch_types=[
            pltpu.VMEM((NW * (NB2 // NS),), i32),
            pltpu.VMEM((NB2 // NS,), i32),
            pltpu.VMEM_SHARED((NB2,), i32),
            pltpu.VMEM((NB2,), i32),
            pltpu.VMEM((BLK,), f32),
            pltpu.VMEM((BLK,), f32),
            pltpu.VMEM((LANES * ST3,), i32),
            pltpu.VMEM((NB3,), i32),
            pltpu.VMEM((LANES,), i32),
            pltpu.SMEM((128,), i32),
            pltpu.SemaphoreType.DMA,
            pltpu.SemaphoreType.DMA,
            pltpu.SemaphoreType.DMA,
        ])
    def k3(x_hbm, h2_hbm, scal2_hbm, h3_hbm, scal3_hbm,
           hv2, redsl, shared, ghv, buf0, buf1, hist_v, red_v, scalv,
           coarse, sem0, sem1, semh):
        wid, sid = ids_()
        bufs, sems = [buf0, buf1], [sem0, sem1]
        cp0 = first_cp(x_hbm, bufs, sems, wid)
        cph = _slice_copy(h2_hbm, hv2, semh, sid, NB2)
        pltpu.sync_copy(scal2_hbm, scalv)
        sv = scalv[...]
        p1 = sv[0]
        s1 = sv[1]
        _zero(hist_v, LANES * ST3)
        cph.wait()
        _global_hist(hv2, redsl, shared, ghv, sid, NB2)
        b2, sw = _scan_desc(ghv, coarse, NB2, jnp.int32(k) - s1)
        p12 = p1 * 2048 + b2
        s2 = s1 + sw

        def binfn(ks):
            return ks, (ks >> 10) == p12

        _hist_blocks(x_hbm, lw, bufs, sems, hist_v, wid, ch, binfn, ST3,
                     cp0, bias=-p12 * 1024)
        _reduce_lanes(hist_v, red_v, NB3, ST3)
        _hist_publish(red_v, h3_hbm, wid, NB3, sem0)
        lanes = lax.iota(i32, LANES)
        scalv[...] = jnp.where(lanes == 0, p12,
                               jnp.where(lanes == 1, s2, 0)).astype(i32)

        @pl.when(wid == 0)
        def _():
            pltpu.sync_copy(scalv, scal3_hbm)

    @functools.partial(
        pl.kernel,
        out_type=jax.ShapeDtypeStruct((N,), f32),
        mesh=mesh,
        compiler_params=pltpu.CompilerParams(needs_layout_passes=False),
        scratch_types=[
            pltpu.VMEM((NW * (NB3 // NS),), i32),
            pltpu.VMEM((NB3 // NS,), i32),
            pltpu.VMEM_SHARED((NB3,), i32),
            pltpu.VMEM((NB3,), i32),
            pltpu.VMEM((NW * (NB3 // NS),), i32),
            pltpu.VMEM((BLK,), f32),
            pltpu.VMEM((BLK,), f32),
            pltpu.VMEM((BLK,), f32),
            pltpu.VMEM((BLK,), f32),
            pltpu.VMEM((LANES,), i32),
            pltpu.SMEM((128,), i32),
            pltpu.SemaphoreType.DMA,
            pltpu.SemaphoreType.DMA,
            pltpu.SemaphoreType.DMA,
            pltpu.SemaphoreType.DMA,
            pltpu.SemaphoreType.DMA,
        ])
    def k4(x_hbm, h3_hbm, scal3_hbm, out_hbm,
           hv2, redsl, shared, ghv, slab, buf0, buf1, ob0, ob1, scalv,
           coarse, sem0, sem1, osem0, osem1, semh):
        wid, sid = ids_()
        bufs, sems = [buf0, buf1], [sem0, sem1]
        obufs, osems = [ob0, ob1], [osem0, osem1]
        cp0 = first_cp(x_hbm, bufs, sems, wid)
        cph = _slice_copy(h3_hbm, hv2, semh, sid, NB3)
        pltpu.sync_copy(scal3_hbm, scalv)
        sv = scalv[...]
        p12 = sv[0]
        s2 = sv[1]
        cph.wait()
        _global_hist(hv2, redsl, shared, ghv, sid, NB3)
        b3, sw3 = _scan_desc(ghv, coarse, NB3, jnp.int32(k) - s2)
        tk = p12 * 1024 + b3
        need = jnp.int32(k) - (s2 + sw3)
        iot = lax.iota(jnp.int32, LANES)
        sl3 = NB3 // NS
        s3 = b3 // sl3
        j3 = b3 - s3 * sl3
        jg = (j3 // LANES) * LANES
        lane = j3 - jg
        scp = pltpu.async_copy(
            h3_hbm.at[pl.ds(s3 * (NW * sl3), NW * sl3)], slab, semh)
        scp.wait()
        e_me = jnp.int32(0)
        base = jnp.int32(0)
        for w in range(NW):
            e_w = jnp.sum(jnp.where(
                iot == lane, slab[pl.ds(w * sl3 + jg, LANES)], 0))
            e_me = jnp.where(w == wid, e_w, e_me)
            base = base + jnp.where(w < wid, e_w, 0)
        m = jnp.maximum(jnp.int32(0), jnp.minimum(need - base, e_me))
        thr = tk - jnp.where(m == e_me, 1, 0).astype(i32)
        thr_bits = jnp.where(thr >= 0, thr, thr ^ jnp.int32(0x7FFFFFFF))
        f_thr = lax.bitcast_convert_type(thr_bits, f32)

        nblk = ch // BLK
        cps = [cp0, None]
        ocps = [None, None]
        for b in range(nblk):
            if b + 1 < nblk:
                cps[(b + 1) % 2] = pltpu.async_copy(
                    _blk(x_hbm, wid * ch + (b + 1) * BLK, lw),
                    bufs[(b + 1) % 2], sems[(b + 1) % 2])
            cps[b % 2].wait()
            if b >= 2:
                ocps[b % 2].wait()
            cur = bufs[b % 2]
            ob = obufs[b % 2]

            def body(i, _):
                outs = []
                for g in range(GRP):
                    slc = pl.ds(i * (GRP * LANES) + g * LANES, LANES)
                    v = cur[slc]
                    outs.append((slc, jnp.where(v > f_thr, v, 0.0)))
                for slc, o in outs:
                    ob[slc] = o
                return 0

            lax.fori_loop(0, BLK // (GRP * LANES), body, 0, unroll=2)
            ocps[b % 2] = pltpu.async_copy(
                ob, _blk(out_hbm, wid * ch + b * BLK, lw), osems[b % 2])
        ocps[(nblk - 2) % 2].wait()
        ocps[(nblk - 1) % 2].wait()

        @pl.when((m > 0) & (m < e_me))
        def _():
            running = jnp.zeros((LANES,), i32)
            for b in range(nblk):
                pltpu.sync_copy(_blk(x_hbm, wid * ch + b * BLK, lw), buf0)

                def fb(i, run):
                    v = buf0[pl.ds(i * LANES, LANES)]
                    ks = _key16(v)
                    eq = ks == tk
                    incl = plsc.cumsum(eq.astype(i32))
                    pc = plsc.all_reduce_population_count(eq)
                    sel = eq & ((incl + run) <= m)
                    ob0[pl.ds(i * LANES, LANES)] = jnp.where(
                        ks > tk, v, jnp.where(sel, v, 0.0))
                    return run + pc

                running = lax.fori_loop(0, BLK // LANES, fb, running)
                pltpu.sync_copy(ob0, _blk(out_hbm, wid * ch + b * BLK, lw))

    return k1, k2, k3, k4


def kernel(x):
    shape = x.shape
    xf = x.reshape(-1)
    N = xf.shape[0]
    k = int(0.01 * N)
    k1, k2, k3, k4 = _make_kernels(shape, k)
    h1 = k1(xf)
    h2, scal2 = k2(xf, h1)
    h3, scal3 = k3(xf, h2, scal2)
    return k4(xf, h3, scal3).reshape(shape)
